# baseline (speedup 1.0000x reference)
import functools

import jax
import jax.numpy as jnp
from jax import lax
from jax.experimental import pallas as pl
from jax.experimental.pallas import tpu as pltpu
from jax.experimental.pallas import tpu_sc as plsc

_WORD_DIM = 64
_POS_DIM = 16
_OUT_DIM = _WORD_DIM + _POS_DIM
_PAD_DIM = 128
_CHUNK = 128
_LANES = 16
_LBLOCK = 8


def _make_lookup(seq_len, batch, pos_vocab):
    info = plsc.get_sparse_core_info()
    num_workers = info.num_cores * info.num_subcores
    n_groups = seq_len // _LBLOCK
    per_w = (n_groups + num_workers - 1) // num_workers
    n_chunks = batch // _CHUNK
    mesh = plsc.VectorSubcoreMesh(core_axis_name="c", subcore_axis_name="s")

    @functools.partial(
        pl.kernel,
        out_type=jax.ShapeDtypeStruct((seq_len, _OUT_DIM, batch),
                                      jnp.float32),
        mesh=mesh,
        scratch_types=[
            pltpu.VMEM((_LBLOCK, batch), jnp.int32),
            pltpu.VMEM((_LBLOCK, batch), jnp.int32),
            pltpu.VMEM((_POS_DIM, pos_vocab), jnp.float32),
            pltpu.VMEM((_CHUNK, _PAD_DIM), jnp.float32),
            pltpu.VMEM((_CHUNK, _PAD_DIM), jnp.float32),
            pltpu.VMEM((_OUT_DIM, _CHUNK), jnp.float32),
            pltpu.SemaphoreType.DMA,
            pltpu.SemaphoreType.DMA,
        ],
        compiler_params=pltpu.CompilerParams(needs_layout_passes=False),
    )
    def lookup(xt_hbm, pt_hbm, wt_hbm, ptab_hbm, out_hbm, xi, pi, ptab_v,
               wbuf0, wbuf1, obuf, sem0, sem1):
        wid = lax.axis_index("s") * info.num_cores + lax.axis_index("c")
        pltpu.sync_copy(ptab_hbm, ptab_v)

        lane = lax.broadcasted_iota(jnp.int32, (_LANES,), 0)

        def fire(j, c, wb, sem):
            pltpu.async_copy(
                wt_hbm.at[xi.at[j, pl.ds(c * _CHUNK, _CHUNK)]], wb, sem)

        def drain(l0, j, c, wb, sem):
            pltpu.make_async_copy(
                wt_hbm.at[xi.at[j, pl.ds(c * _CHUNK, _CHUNK)]], wb, sem
            ).wait()

            @pl.loop(0, _CHUNK // _LANES)
            def _(v):
                tok = lane + v * _LANES
                posv = pi[j, pl.ds(c * _CHUNK + v * _LANES, _LANES)]
                for f in range(_WORD_DIM):
                    obuf[f, pl.ds(v * _LANES, _LANES)] = plsc.load_gather(
                        wb, [tok, jnp.full((_LANES,), f, jnp.int32)])
                for f in range(_POS_DIM):
                    obuf[_WORD_DIM + f,
                         pl.ds(v * _LANES, _LANES)] = plsc.load_gather(
                             ptab_v, [jnp.full((_LANES,), f, jnp.int32),
                                      posv])

            pltpu.sync_copy(
                obuf,
                out_hbm.at[l0 + j, :, pl.ds(c * _CHUNK, _CHUNK)])

        @pl.loop(0, per_w)
        def _(k):
            grp = k * num_workers + wid

            @pl.when(grp < n_groups)
            def _():
                l0 = pl.multiple_of(grp * _LBLOCK, _LBLOCK)
                pltpu.sync_copy(xt_hbm.at[pl.ds(l0, _LBLOCK)], xi)
                pltpu.sync_copy(pt_hbm.at[pl.ds(l0, _LBLOCK)], pi)

                fire(0, 0, wbuf0, sem0)

                @pl.loop(0, _LBLOCK * n_chunks // 2)
                def _(t):
                    i0 = 2 * t
                    j0, c0 = i0 // n_chunks, i0 % n_chunks
                    i1 = i0 + 1
                    j1, c1 = i1 // n_chunks, i1 % n_chunks
                    fire(j1, c1, wbuf1, sem1)
                    drain(l0, j0, c0, wbuf0, sem0)

                    @pl.when(i1 + 1 < _LBLOCK * n_chunks)
                    def _():
                        i2 = i1 + 1
                        fire(i2 // n_chunks, i2 % n_chunks, wbuf0, sem0)

                    drain(l0, j1, c1, wbuf1, sem1)

    return lookup


def kernel(x, pos, word_table, pos_table):
    b, l = x.shape
    wt_pad = jnp.pad(word_table, ((0, 0), (0, _PAD_DIM - _WORD_DIM)))
    lookup = _make_lookup(l, b, pos_table.shape[0])
    out = lookup(x.T, pos.T, wt_pad, pos_table.T)
    return out.transpose(2, 0, 1)


# speedup vs baseline: 1.0270x; 1.0270x over previous
import functools

import jax
import jax.numpy as jnp
from jax import lax
from jax.experimental import pallas as pl
from jax.experimental.pallas import tpu as pltpu
from jax.experimental.pallas import tpu_sc as plsc

_WORD_DIM = 64
_POS_DIM = 16
_OUT_DIM = _WORD_DIM + _POS_DIM
_PAD_DIM = 128
_CHUNK = 128
_LANES = 16
_LBLOCK = 8


def _make_lookup(seq_len, batch, pos_vocab):
    info = plsc.get_sparse_core_info()
    num_workers = info.num_cores * info.num_subcores
    n_groups = seq_len // _LBLOCK
    per_w = (n_groups + num_workers - 1) // num_workers
    n_chunks = batch // _CHUNK
    mesh = plsc.VectorSubcoreMesh(core_axis_name="c", subcore_axis_name="s")

    @functools.partial(
        pl.kernel,
        out_type=jax.ShapeDtypeStruct((seq_len, _OUT_DIM, batch),
                                      jnp.float32),
        mesh=mesh,
        scratch_types=[
            pltpu.VMEM((_LBLOCK, batch), jnp.int32),
            pltpu.VMEM((_LBLOCK, batch), jnp.int32),
            pltpu.VMEM((pos_vocab, _POS_DIM), jnp.float32),
            pltpu.VMEM((_CHUNK, _PAD_DIM), jnp.float32),
            pltpu.VMEM((_CHUNK, _PAD_DIM), jnp.float32),
            pltpu.VMEM((_OUT_DIM, _CHUNK), jnp.float32),
            pltpu.SemaphoreType.DMA,
            pltpu.SemaphoreType.DMA,
        ],
        compiler_params=pltpu.CompilerParams(needs_layout_passes=False),
    )
    def lookup(xt_hbm, pt_hbm, wt_hbm, ptab_hbm, out_hbm, xi, pi, ptab_v,
               wbuf0, wbuf1, obuf, sem0, sem1):
        wid = lax.axis_index("s") * info.num_cores + lax.axis_index("c")
        pltpu.sync_copy(ptab_hbm, ptab_v)

        lane = lax.broadcasted_iota(jnp.int32, (_LANES,), 0)

        def fire(j, c, wb, sem):
            pltpu.async_copy(
                wt_hbm.at[xi.at[j, pl.ds(c * _CHUNK, _CHUNK)]], wb, sem)

        def drain(l0, j, c, wb, sem):
            pltpu.make_async_copy(
                wt_hbm.at[xi.at[j, pl.ds(c * _CHUNK, _CHUNK)]], wb, sem
            ).wait()

            @pl.loop(0, _CHUNK // _LANES)
            def _(v):
                posv = pi[j, pl.ds(c * _CHUNK + v * _LANES, _LANES)]
                for u in range(_LANES):
                    tok = v * _LANES + u
                    tokv = jnp.full((_LANES,), tok, jnp.int32)
                    for f0 in range(0, _WORD_DIM, _LANES):
                        plsc.store_scatter(
                            obuf, [lane + f0, tokv],
                            wb[tok, pl.ds(f0, _LANES)])
                    plsc.store_scatter(
                        obuf, [lane + _WORD_DIM, tokv],
                        ptab_v[posv[u]])

            pltpu.sync_copy(
                obuf,
                out_hbm.at[l0 + j, :, pl.ds(c * _CHUNK, _CHUNK)])

        @pl.loop(0, per_w)
        def _(k):
            grp = k * num_workers + wid

            @pl.when(grp < n_groups)
            def _():
                l0 = pl.multiple_of(grp * _LBLOCK, _LBLOCK)
                pltpu.sync_copy(xt_hbm.at[pl.ds(l0, _LBLOCK)], xi)
                pltpu.sync_copy(pt_hbm.at[pl.ds(l0, _LBLOCK)], pi)

                fire(0, 0, wbuf0, sem0)

                @pl.loop(0, _LBLOCK * n_chunks // 2)
                def _(t):
                    i0 = 2 * t
                    j0, c0 = i0 // n_chunks, i0 % n_chunks
                    i1 = i0 + 1
                    j1, c1 = i1 // n_chunks, i1 % n_chunks
                    fire(j1, c1, wbuf1, sem1)
                    drain(l0, j0, c0, wbuf0, sem0)

                    @pl.when(i1 + 1 < _LBLOCK * n_chunks)
                    def _():
                        i2 = i1 + 1
                        fire(i2 // n_chunks, i2 % n_chunks, wbuf0, sem0)

                    drain(l0, j1, c1, wbuf1, sem1)

    return lookup


def kernel(x, pos, word_table, pos_table):
    b, l = x.shape
    wt_pad = jnp.pad(word_table, ((0, 0), (0, _PAD_DIM - _WORD_DIM)))
    lookup = _make_lookup(l, b, pos_table.shape[0])
    out = lookup(x.T, pos.T, wt_pad, pos_table)
    return out.transpose(2, 0, 1)


# speedup vs baseline: 1.4310x; 1.3933x over previous
import functools

import jax
import jax.numpy as jnp
from jax import lax
from jax.experimental import pallas as pl
from jax.experimental.pallas import tpu as pltpu
from jax.experimental.pallas import tpu_sc as plsc

_WORD_DIM = 64
_POS_DIM = 16
_OUT_DIM = _WORD_DIM + _POS_DIM
_PAD_DIM = 128
_CHUNK = 128
_LANES = 16


def _make_lookup(n_rows, pos_vocab):
    info = plsc.get_sparse_core_info()
    num_workers = info.num_cores * info.num_subcores
    per_w = n_rows // num_workers
    n_chunks = per_w // _CHUNK
    assert n_chunks % 2 == 0
    mesh = plsc.VectorSubcoreMesh(core_axis_name="c", subcore_axis_name="s")

    @functools.partial(
        pl.kernel,
        out_type=jax.ShapeDtypeStruct((n_rows, _OUT_DIM), jnp.float32),
        mesh=mesh,
        scratch_types=[
            pltpu.VMEM((per_w,), jnp.int32),
            pltpu.VMEM((per_w,), jnp.int32),
            pltpu.VMEM((pos_vocab, _POS_DIM), jnp.float32),
            pltpu.VMEM((_CHUNK, _PAD_DIM), jnp.float32),
            pltpu.VMEM((_CHUNK, _PAD_DIM), jnp.float32),
            pltpu.VMEM((_CHUNK, _OUT_DIM), jnp.float32),
            pltpu.SemaphoreType.DMA,
            pltpu.SemaphoreType.DMA,
        ],
    )
    def lookup(x_hbm, p_hbm, wt_hbm, pt_hbm, out_hbm, xi, pi, pt_v,
               wbuf0, wbuf1, obuf, sem0, sem1):
        wid = lax.axis_index("s") * info.num_cores + lax.axis_index("c")
        base = wid * per_w
        pltpu.sync_copy(x_hbm.at[pl.ds(base, per_w)], xi)
        pltpu.sync_copy(p_hbm.at[pl.ds(base, per_w)], pi)
        pltpu.sync_copy(pt_hbm, pt_v)

        def fire(g, wb, sem):
            pltpu.async_copy(
                wt_hbm.at[xi.at[pl.ds(g * _CHUNK, _CHUNK)]], wb, sem)

        def drain(g, wb, sem):
            off = g * _CHUNK
            pltpu.make_async_copy(
                wt_hbm.at[xi.at[pl.ds(off, _CHUNK)]], wb, sem).wait()

            @pl.loop(0, _CHUNK // _LANES)
            def _(v):
                posv = pi[pl.ds(off + v * _LANES, _LANES)]
                for j in range(_LANES):
                    row = v * _LANES + j
                    for c in range(_WORD_DIM // _LANES):
                        obuf[row, pl.ds(c * _LANES, _LANES)] = (
                            wb[row, pl.ds(c * _LANES, _LANES)])
                    obuf[row, pl.ds(_WORD_DIM, _POS_DIM)] = pt_v[posv[j]]

            pltpu.sync_copy(obuf, out_hbm.at[pl.ds(base + off, _CHUNK)])

        fire(0, wbuf0, sem0)

        @pl.loop(0, n_chunks // 2)
        def _(t):
            g = 2 * t
            fire(g + 1, wbuf1, sem1)
            drain(g, wbuf0, sem0)

            @pl.when(g + 2 < n_chunks)
            def _():
                fire(g + 2, wbuf0, sem0)

            drain(g + 1, wbuf1, sem1)

    return lookup


def kernel(x, pos, word_table, pos_table):
    b, l = x.shape
    n_rows = b * l
    wt_pad = jnp.pad(word_table.T, ((0, _PAD_DIM - _WORD_DIM), (0, 0))).T
    lookup = _make_lookup(n_rows, pos_table.shape[0])
    out = lookup(x.reshape(n_rows), pos.reshape(n_rows), wt_pad, pos_table)
    return out.reshape(b, l, _OUT_DIM)


# speedup vs baseline: 1.4390x; 1.0056x over previous
import functools

import jax
import jax.numpy as jnp
from jax import lax
from jax.experimental import pallas as pl
from jax.experimental.pallas import tpu as pltpu
from jax.experimental.pallas import tpu_sc as plsc

_WORD_DIM = 64
_POS_DIM = 16
_OUT_DIM = _WORD_DIM + _POS_DIM
_PAD_DIM = 128
_CHUNK = 128
_LANES = 16
_NBUF = 4


def _make_lookup(n_rows, pos_vocab):
    info = plsc.get_sparse_core_info()
    num_workers = info.num_cores * info.num_subcores
    per_w = n_rows // num_workers
    n_chunks = per_w // _CHUNK
    mesh = plsc.VectorSubcoreMesh(core_axis_name="c", subcore_axis_name="s")

    @functools.partial(
        pl.kernel,
        out_type=jax.ShapeDtypeStruct((n_rows, _OUT_DIM), jnp.float32),
        mesh=mesh,
        scratch_types=[
            pltpu.VMEM((per_w,), jnp.int32),
            pltpu.VMEM((per_w,), jnp.int32),
            pltpu.VMEM((pos_vocab, _POS_DIM), jnp.float32),
            pltpu.VMEM((_NBUF, _CHUNK, _PAD_DIM), jnp.float32),
            pltpu.VMEM((_CHUNK, _OUT_DIM), jnp.float32),
        ] + [pltpu.SemaphoreType.DMA] * _NBUF,
    )
    def lookup(x_hbm, p_hbm, wt_hbm, pt_hbm, out_hbm, xi, pi, pt_v,
               wbuf, obuf, *sems):
        wid = lax.axis_index("s") * info.num_cores + lax.axis_index("c")
        base = wid * per_w
        pltpu.sync_copy(x_hbm.at[pl.ds(base, per_w)], xi)
        pltpu.sync_copy(p_hbm.at[pl.ds(base, per_w)], pi)
        pltpu.sync_copy(pt_hbm, pt_v)

        def fire(g, p):
            pltpu.async_copy(
                wt_hbm.at[xi.at[pl.ds(g * _CHUNK, _CHUNK)]], wbuf.at[p],
                sems[p])

        def drain(g, p):
            off = g * _CHUNK
            pltpu.make_async_copy(
                wt_hbm.at[xi.at[pl.ds(0, _CHUNK)]], wbuf.at[p],
                sems[p]).wait()
            wb = wbuf.at[p]

            @pl.loop(0, _CHUNK // _LANES)
            def _(v):
                posv = pi[pl.ds(off + v * _LANES, _LANES)]
                for j in range(_LANES):
                    row = v * _LANES + j
                    for c in range(_WORD_DIM // _LANES):
                        obuf[row, pl.ds(c * _LANES, _LANES)] = (
                            wb[row, pl.ds(c * _LANES, _LANES)])
                    obuf[row, pl.ds(_WORD_DIM, _POS_DIM)] = pt_v[posv[j]]

            pltpu.sync_copy(obuf, out_hbm.at[pl.ds(base + off, _CHUNK)])

        for p in range(_NBUF - 1):
            fire(p, p)

        @pl.loop(0, (n_chunks + _NBUF - 1) // _NBUF)
        def _(t):
            g0 = t * _NBUF
            for p in range(_NBUF):
                g = g0 + p

                @pl.when(g + _NBUF - 1 < n_chunks)
                def _():
                    fire(g + _NBUF - 1, (p + _NBUF - 1) % _NBUF)

                @pl.when(g < n_chunks)
                def _():
                    drain(g, p)

    return lookup


def kernel(x, pos, word_table, pos_table):
    b, l = x.shape
    n_rows = b * l
    wt_pad = jnp.pad(word_table.T, ((0, _PAD_DIM - _WORD_DIM), (0, 0))).T
    lookup = _make_lookup(n_rows, pos_table.shape[0])
    out = lookup(x.reshape(n_rows), pos.reshape(n_rows), wt_pad, pos_table)
    return out.reshape(b, l, _OUT_DIM)


# speedup vs baseline: 1.4424x; 1.0024x over previous
import functools

import jax
import jax.numpy as jnp
from jax import lax
from jax.experimental import pallas as pl
from jax.experimental.pallas import tpu as pltpu
from jax.experimental.pallas import tpu_sc as plsc

_WORD_DIM = 64
_POS_DIM = 16
_OUT_DIM = _WORD_DIM + _POS_DIM
_PAD_DIM = 128
_CHUNK = 128
_LANES = 16
_NBUF = 4


def _make_lookup(n_rows, pos_vocab):
    info = plsc.get_sparse_core_info()
    num_workers = info.num_cores * info.num_subcores
    per_w = n_rows // num_workers
    n_chunks = per_w // _CHUNK
    mesh = plsc.VectorSubcoreMesh(core_axis_name="c", subcore_axis_name="s")

    @functools.partial(
        pl.kernel,
        out_type=jax.ShapeDtypeStruct((n_rows, _OUT_DIM), jnp.float32),
        mesh=mesh,
        scratch_types=[
            pltpu.VMEM((per_w,), jnp.int32),
            pltpu.VMEM((per_w,), jnp.int32),
            pltpu.VMEM((pos_vocab, _POS_DIM), jnp.float32),
            pltpu.VMEM((_NBUF, _CHUNK, _PAD_DIM), jnp.float32),
            pltpu.VMEM((_CHUNK, _OUT_DIM), jnp.float32),
        ] + [pltpu.SemaphoreType.DMA] * _NBUF,
    )
    def lookup(x_hbm, p_hbm, wt_hbm, pt_hbm, out_hbm, xi, pi, pt_v,
               wbuf, obuf, *sems):
        wid = lax.axis_index("s") * info.num_cores + lax.axis_index("c")
        base = wid * per_w
        pltpu.sync_copy(x_hbm.at[pl.ds(base, per_w)], xi)
        pltpu.sync_copy(p_hbm.at[pl.ds(base, per_w)], pi)
        pltpu.sync_copy(pt_hbm, pt_v)

        def fire(g, p):
            pltpu.async_copy(
                wt_hbm.at[xi.at[pl.ds(g * _CHUNK, _CHUNK)]], wbuf.at[p],
                sems[p])

        def drain(g, p):
            off = g * _CHUNK
            pltpu.make_async_copy(
                wt_hbm.at[xi.at[pl.ds(off, _CHUNK)]], wbuf.at[p],
                sems[p]).wait()
            wb = wbuf.at[p]

            @pl.loop(0, _CHUNK // _LANES)
            def _(v):
                posv = pi[pl.ds(off + v * _LANES, _LANES)]
                for j in range(_LANES):
                    row = v * _LANES + j
                    for c in range(_WORD_DIM // _LANES):
                        obuf[row, pl.ds(c * _LANES, _LANES)] = (
                            wb[row, pl.ds(c * _LANES, _LANES)])
                    obuf[row, pl.ds(_WORD_DIM, _POS_DIM)] = pt_v[posv[j]]

            pltpu.sync_copy(obuf, out_hbm.at[pl.ds(base + off, _CHUNK)])

        for p in range(_NBUF - 1):
            fire(p, p)

        @pl.loop(0, (n_chunks + _NBUF - 1) // _NBUF)
        def _(t):
            g0 = t * _NBUF
            for p in range(_NBUF):
                g = g0 + p

                @pl.when(g + _NBUF - 1 < n_chunks)
                def _():
                    fire(g + _NBUF - 1, (p + _NBUF - 1) % _NBUF)

                @pl.when(g < n_chunks)
                def _():
                    drain(g, p)

    return lookup


def kernel(x, pos, word_table, pos_table):
    b, l = x.shape
    n_rows = b * l
    wt_pad = jnp.pad(word_table.T, ((0, _PAD_DIM - _WORD_DIM), (0, 0))).T
    lookup = _make_lookup(n_rows, pos_table.shape[0])
    out = lookup(x.reshape(n_rows), pos.reshape(n_rows), wt_pad, pos_table)
    return out.reshape(b, l, _OUT_DIM)
